# baseline (speedup 1.0000x reference)
import functools

import jax
import jax.numpy as jnp
from jax import lax
from jax.experimental import pallas as pl
from jax.experimental.pallas import tpu as pltpu
from jax.experimental.pallas import tpu_sc as plsc

N = 10000
E = 320000
D = 128
HID = 128
OUT = 256
NLAYER = 2
NMASK = 1000

NC = 2
NS = 16

_EPT = E // NS
_CH = 80
_NCH = _EPT // _CH
_NPASS = 10
_PP = _NCH // _NPASS
_RPT = 624
_RPT_LAST = N - (NS - 1) * _RPT

_f32 = jnp.float32
_HIGH = jax.lax.Precision.HIGHEST


@functools.lru_cache(maxsize=None)
def _get_sc_agg_pair():
    mesh = plsc.VectorSubcoreMesh(core_axis_name="c", subcore_axis_name="s")

    @functools.partial(
        pl.kernel,
        mesh=mesh,
        out_type=jax.ShapeDtypeStruct((NC, N, D), _f32),
        scratch_types=[
            pltpu.VMEM((_PP * _CH,), jnp.int32),
            pltpu.VMEM((_PP, _CH), jnp.int32),
            pltpu.VMEM((_CH, D), _f32),
            pltpu.VMEM((_CH, D), _f32),
            pltpu.VMEM((_CH, D), _f32),
            pltpu.VMEM((_CH, D), _f32),
            pltpu.VMEM_SHARED((N, D), _f32),
            pltpu.SemaphoreType.DMA,
            pltpu.SemaphoreType.DMA,
            pltpu.SemaphoreType.DMA,
            pltpu.SemaphoreType.DMA,
            pltpu.SemaphoreType.DMA,
            pltpu.SemaphoreType.DMA,
            pltpu.SemaphoreType.DMA,
            pltpu.SemaphoreType.DMA,
        ],
    )
    def _sc_agg_pair(h_on, h_tg, src, dst4, zrows, out,
                     sidx, didx, b0, b1, b2, b3, acc_sh,
                     sg0, sg1, sg2, sg3, ss0, ss1, ss2, ss3):
        c = lax.axis_index("c")
        s = lax.axis_index("s")

        @pl.when(s < NS - 1)
        def _():
            pltpu.sync_copy(zrows.at[pl.ds(0, _RPT)],
                            acc_sh.at[pl.ds(s * _RPT, _RPT)])

        @pl.when(s == NS - 1)
        def _():
            pltpu.sync_copy(zrows,
                            acc_sh.at[pl.ds((NS - 1) * _RPT, _RPT_LAST)])

        plsc.subcore_barrier()

        def run(table):
            bufs = (b0, b1, b2, b3)
            gsems = (sg0, sg1, sg2, sg3)
            ssems = (ss0, ss1, ss2, ss3)

            def g_start(j, b):
                pltpu.async_copy(table.at[sidx.at[pl.ds(j * _CH, _CH)]],
                                 bufs[b], gsems[b])

            def g_wait(j, b):
                pltpu.make_async_copy(table.at[sidx.at[pl.ds(j * _CH, _CH)]],
                                      bufs[b], gsems[b]).wait()

            def s_start(j, b):
                pltpu.async_copy(bufs[b], acc_sh.at[didx.at[j]], ssems[b],
                                 add=True)

            def s_wait(j, b):
                pltpu.make_async_copy(bufs[b], acc_sh.at[didx.at[j]],
                                      ssems[b]).wait()

            def pass_body(p, carry):
                pltpu.sync_copy(
                    src.at[pl.ds(s * _EPT + p * (_PP * _CH), _PP * _CH)],
                    sidx)
                pltpu.sync_copy(dst4.at[s, p], didx)

                for b in range(4):
                    g_start(b, b)

                def quad(t, carry2):
                    j = 4 * t
                    for b in range(4):
                        @pl.when(j + b < _PP)
                        def _(b=b):
                            g_wait(j + b, b)
                            s_start(j + b, b)
                    for b in range(4):
                        @pl.when(j + 4 + b < _PP)
                        def _(b=b):
                            s_wait(j + b, b)
                            g_start(j + 4 + b, b)
                    return carry2

                lax.fori_loop(0, (_PP + 3) // 4, quad, 0)

                for b in range(4):
                    s_wait(0, b)
                return carry

            lax.fori_loop(0, _NPASS, pass_body, 0)

        @pl.when(c == 0)
        def _():
            run(h_on)

        @pl.when(c == 1)
        def _():
            run(h_tg)

        plsc.subcore_barrier()

        @pl.when(s < NS - 1)
        def _():
            pltpu.sync_copy(acc_sh.at[pl.ds(s * _RPT, _RPT)],
                            out.at[c].at[pl.ds(s * _RPT, _RPT)])

        @pl.when(s == NS - 1)
        def _():
            pltpu.sync_copy(acc_sh.at[pl.ds((NS - 1) * _RPT, _RPT_LAST)],
                            out.at[c].at[pl.ds((NS - 1) * _RPT, _RPT_LAST)])

    return _sc_agg_pair


_GCH = 32


@functools.lru_cache(maxsize=None)
def _get_sc_mask_gather():
    mesh = plsc.VectorSubcoreMesh(core_axis_name="c", subcore_axis_name="s")

    @functools.partial(
        pl.kernel,
        mesh=mesh,
        out_type=[jax.ShapeDtypeStruct((NMASK, D), _f32),
                  jax.ShapeDtypeStruct((NMASK, D), _f32)],
        scratch_types=[
            pltpu.VMEM((_GCH,), jnp.int32),
            pltpu.VMEM((_GCH, D), _f32),
            pltpu.VMEM((_GCH, D), _f32),
            pltpu.SemaphoreType.DMA,
        ],
    )
    def _sc_mask_gather(h_on, h_tg, mask, out_a, out_b, idx, ra, rb, sem):
        c = lax.axis_index("c")
        s = lax.axis_index("s")
        w = s * NC + c
        base = jnp.where(w * _GCH + _GCH <= NMASK, w * _GCH, NMASK - _GCH)
        pltpu.sync_copy(mask.at[pl.ds(base, _GCH)], idx)
        pltpu.async_copy(h_on.at[idx], ra, sem).wait()
        pltpu.async_copy(h_tg.at[idx], rb, sem).wait()
        pltpu.sync_copy(ra, out_a.at[pl.ds(base, _GCH)])
        pltpu.sync_copy(rb, out_b.at[pl.ds(base, _GCH)])

    return _sc_mask_gather


_RM = 1000


def _mask_body(x_ref, mask_ref, tok_ref, o_ref):
    i = pl.program_id(0)
    rows = lax.broadcasted_iota(jnp.int32, (_RM, 1), 0) + i * _RM
    hit = jnp.any(rows == mask_ref[...], axis=1, keepdims=True)
    o_ref[...] = jnp.where(hit, tok_ref[...], x_ref[...])


def _mask_call(x, mask_nodes, mask_token):
    return pl.pallas_call(
        _mask_body,
        grid=(N // _RM,),
        in_specs=[
            pl.BlockSpec((_RM, D), lambda i: (i, 0)),
            pl.BlockSpec((1, NMASK), lambda i: (0, 0)),
            pl.BlockSpec((1, D), lambda i: (0, 0)),
        ],
        out_specs=pl.BlockSpec((_RM, D), lambda i: (i, 0)),
        out_shape=jax.ShapeDtypeStruct((N, D), _f32),
    )(x, mask_nodes.reshape(1, NMASK), mask_token)


_R = 1000
_G = N // _R


def _stage_a_body(hA, hB, aA, aB, wA, wB,
                  zA, zB, s1A, s2A, s1B, s2B):
    i = pl.program_id(0)

    @pl.when(i == 0)
    def _():
        s1A[...] = jnp.zeros_like(s1A)
        s2A[...] = jnp.zeros_like(s2A)
        s1B[...] = jnp.zeros_like(s1B)
        s2B[...] = jnp.zeros_like(s2B)

    def one(h, a, w, z, s1, s2):
        m = h[...] + a[0]
        zv = jnp.dot(m, w[...], preferred_element_type=_f32, precision=_HIGH)
        z[...] = zv
        s1[...] += jnp.sum(zv, axis=0, keepdims=True)
        s2[...] += jnp.sum(zv * zv, axis=0, keepdims=True)

    one(hA, aA, wA, zA, s1A, s2A)
    one(hB, aB, wB, zB, s1B, s2B)


def _stage_a(hA, hB, agg, wA, wB):
    outs = pl.pallas_call(
        _stage_a_body,
        grid=(_G,),
        in_specs=[
            pl.BlockSpec((_R, HID), lambda i: (i, 0)),
            pl.BlockSpec((_R, HID), lambda i: (i, 0)),
            pl.BlockSpec((1, _R, HID), lambda i: (0, i, 0)),
            pl.BlockSpec((1, _R, HID), lambda i: (1, i, 0)),
            pl.BlockSpec((HID, OUT), lambda i: (0, 0)),
            pl.BlockSpec((HID, OUT), lambda i: (0, 0)),
        ],
        out_specs=[
            pl.BlockSpec((_R, OUT), lambda i: (i, 0)),
            pl.BlockSpec((_R, OUT), lambda i: (i, 0)),
            pl.BlockSpec((1, OUT), lambda i: (0, 0)),
            pl.BlockSpec((1, OUT), lambda i: (0, 0)),
            pl.BlockSpec((1, OUT), lambda i: (0, 0)),
            pl.BlockSpec((1, OUT), lambda i: (0, 0)),
        ],
        out_shape=[
            jax.ShapeDtypeStruct((N, OUT), _f32),
            jax.ShapeDtypeStruct((N, OUT), _f32),
            jax.ShapeDtypeStruct((1, OUT), _f32),
            jax.ShapeDtypeStruct((1, OUT), _f32),
            jax.ShapeDtypeStruct((1, OUT), _f32),
            jax.ShapeDtypeStruct((1, OUT), _f32),
        ],
        compiler_params=pltpu.CompilerParams(
            dimension_semantics=("arbitrary",)),
    )(hA, hB, agg, agg, wA, wB)
    return outs


def _bn_scale_shift(s1, s2, g, b, n):
    mu = s1 / n
    var = s2 / n - mu * mu
    scale = g / jnp.sqrt(var + 1e-5)
    shift = b - mu * scale
    return scale, shift


def _stage_b_body(zA, zB, s1A, s2A, s1B, s2B, gA, bA, gB, bB, wA, wB,
                  yA, yB, t1A, t2A, t1B, t2B):
    i = pl.program_id(0)

    @pl.when(i == 0)
    def _():
        t1A[...] = jnp.zeros_like(t1A)
        t2A[...] = jnp.zeros_like(t2A)
        t1B[...] = jnp.zeros_like(t1B)
        t2B[...] = jnp.zeros_like(t2B)

    def one(z, s1, s2, g, b, w, y, t1, t2):
        scale, shift = _bn_scale_shift(s1[...], s2[...], g[...], b[...], N)
        zn = jnp.maximum(z[...] * scale + shift, 0.0)
        yv = jnp.dot(zn, w[...], preferred_element_type=_f32, precision=_HIGH)
        y[...] = yv
        t1[...] += jnp.sum(yv, axis=0, keepdims=True)
        t2[...] += jnp.sum(yv * yv, axis=0, keepdims=True)

    one(zA, s1A, s2A, gA, bA, wA, yA, t1A, t2A)
    one(zB, s1B, s2B, gB, bB, wB, yB, t1B, t2B)


def _stage_b(zA, zB, s1A, s2A, s1B, s2B, gA, bA, gB, bB, wA, wB):
    vec = lambda: pl.BlockSpec((1, OUT), lambda i: (0, 0))
    outs = pl.pallas_call(
        _stage_b_body,
        grid=(_G,),
        in_specs=[
            pl.BlockSpec((_R, OUT), lambda i: (i, 0)),
            pl.BlockSpec((_R, OUT), lambda i: (i, 0)),
            vec(), vec(), vec(), vec(),
            vec(), vec(), vec(), vec(),
            pl.BlockSpec((OUT, HID), lambda i: (0, 0)),
            pl.BlockSpec((OUT, HID), lambda i: (0, 0)),
        ],
        out_specs=[
            pl.BlockSpec((_R, HID), lambda i: (i, 0)),
            pl.BlockSpec((_R, HID), lambda i: (i, 0)),
            pl.BlockSpec((1, HID), lambda i: (0, 0)),
            pl.BlockSpec((1, HID), lambda i: (0, 0)),
            pl.BlockSpec((1, HID), lambda i: (0, 0)),
            pl.BlockSpec((1, HID), lambda i: (0, 0)),
        ],
        out_shape=[
            jax.ShapeDtypeStruct((N, HID), _f32),
            jax.ShapeDtypeStruct((N, HID), _f32),
            jax.ShapeDtypeStruct((1, HID), _f32),
            jax.ShapeDtypeStruct((1, HID), _f32),
            jax.ShapeDtypeStruct((1, HID), _f32),
            jax.ShapeDtypeStruct((1, HID), _f32),
        ],
        compiler_params=pltpu.CompilerParams(
            dimension_semantics=("arbitrary",)),
    )(zA, zB, s1A, s2A, s1B, s2B, gA, bA, gB, bB, wA, wB)
    return outs


def _stage_c_body(yA, yB, t1A, t2A, t1B, t2B, gA, bA, gB, bB, hA, hB):
    def one(y, t1, t2, g, b, h):
        scale, shift = _bn_scale_shift(t1[...], t2[...], g[...], b[...], N)
        h[...] = jnp.maximum(y[...] * scale + shift, 0.0)

    one(yA, t1A, t2A, gA, bA, hA)
    one(yB, t1B, t2B, gB, bB, hB)


def _stage_c(yA, yB, t1A, t2A, t1B, t2B, gA, bA, gB, bB):
    vec = lambda: pl.BlockSpec((1, HID), lambda i: (0, 0))
    return pl.pallas_call(
        _stage_c_body,
        grid=(_G,),
        in_specs=[
            pl.BlockSpec((_R, HID), lambda i: (i, 0)),
            pl.BlockSpec((_R, HID), lambda i: (i, 0)),
            vec(), vec(), vec(), vec(),
            vec(), vec(), vec(), vec(),
        ],
        out_specs=[
            pl.BlockSpec((_R, HID), lambda i: (i, 0)),
            pl.BlockSpec((_R, HID), lambda i: (i, 0)),
        ],
        out_shape=[
            jax.ShapeDtypeStruct((N, HID), _f32),
            jax.ShapeDtypeStruct((N, HID), _f32),
        ],
    )(yA, yB, t1A, t2A, t1B, t2B, gA, bA, gB, bB)


def _loss_body(a_ref, b_ref, o_ref):
    a = a_ref[...]
    b = b_ref[...]
    ab = jnp.sum(a * b, axis=1, keepdims=True)
    na = jnp.sqrt(jnp.sum(a * a, axis=1, keepdims=True)) + 1e-12
    nb = jnp.sqrt(jnp.sum(b * b, axis=1, keepdims=True)) + 1e-12
    t = 1.0 - ab / (na * nb)
    o_ref[0, 0] = jnp.sum(t) * (1.0 / NMASK)


def _loss_call(a, b):
    out = pl.pallas_call(
        _loss_body,
        in_specs=[
            pl.BlockSpec((NMASK, D), lambda: (0, 0)),
            pl.BlockSpec((NMASK, D), lambda: (0, 0)),
        ],
        out_specs=pl.BlockSpec(memory_space=pltpu.SMEM),
        out_shape=jax.ShapeDtypeStruct((1, 1), _f32),
    )(a, b)
    return out


def kernel(x, edge_index, mask_nodes, mask_token,
           onW1, onW2, onG1, onB1, onGo, onBo,
           tgW1, tgW2, tgG1, tgB1, tgGo, tgBo):
    src = edge_index[0]
    dst = edge_index[1]
    dst4 = dst.reshape(NS, _NPASS, _PP, _CH)
    zrows = jnp.zeros((_RPT_LAST, D), _f32)

    hA = _mask_call(x, mask_nodes, mask_token)
    hB = x

    for i in range(NLAYER):
        agg = _get_sc_agg_pair()(hA, hB, src, dst4, zrows)
        zA, zB, s1A, s2A, s1B, s2B = _stage_a(hA, hB, agg, onW1[i], tgW1[i])
        yA, yB, t1A, t2A, t1B, t2B = _stage_b(
            zA, zB, s1A, s2A, s1B, s2B,
            onG1[i].reshape(1, OUT), onB1[i].reshape(1, OUT),
            tgG1[i].reshape(1, OUT), tgB1[i].reshape(1, OUT),
            onW2[i], tgW2[i])
        hA, hB = _stage_c(
            yA, yB, t1A, t2A, t1B, t2B,
            onGo[i].reshape(1, HID), onBo[i].reshape(1, HID),
            tgGo[i].reshape(1, HID), tgBo[i].reshape(1, HID))

    a, b = _get_sc_mask_gather()(hA, hB, mask_nodes)
    loss = _loss_call(a, b)
    return jnp.reshape(loss, ())


# speedup vs baseline: 1.0437x; 1.0437x over previous
import functools

import jax
import jax.numpy as jnp
from jax import lax
from jax.experimental import pallas as pl
from jax.experimental.pallas import tpu as pltpu
from jax.experimental.pallas import tpu_sc as plsc

N = 10000
E = 320000
D = 128
HID = 128
OUT = 256
NLAYER = 2
NMASK = 1000

NC = 2
NS = 16

_EPC = E // NC
_EPT = _EPC // NS
_CH = 80
_NCH = _EPT // _CH
_NPASS = 5
_PP = _NCH // _NPASS
_RPT = 624
_RPT_LAST = N - (NS - 1) * _RPT

_f32 = jnp.float32
_HIGH = jax.lax.Precision.HIGHEST


@functools.lru_cache(maxsize=None)
def _get_sc_agg():
    mesh = plsc.VectorSubcoreMesh(core_axis_name="c", subcore_axis_name="s")

    @functools.partial(
        pl.kernel,
        mesh=mesh,
        out_type=jax.ShapeDtypeStruct((NC, N, D), _f32),
        scratch_types=[
            pltpu.VMEM((_PP * _CH,), jnp.int32),
            pltpu.VMEM((_PP, _CH), jnp.int32),
            pltpu.VMEM((_CH, D), _f32),
            pltpu.VMEM((_CH, D), _f32),
            pltpu.VMEM((_CH, D), _f32),
            pltpu.VMEM((_CH, D), _f32),
            pltpu.VMEM_SHARED((N, D), _f32),
            pltpu.SemaphoreType.DMA,
            pltpu.SemaphoreType.DMA,
            pltpu.SemaphoreType.DMA,
            pltpu.SemaphoreType.DMA,
            pltpu.SemaphoreType.DMA,
            pltpu.SemaphoreType.DMA,
            pltpu.SemaphoreType.DMA,
            pltpu.SemaphoreType.DMA,
        ],
    )
    def _sc_agg(h, src, dst5, zrows, out,
                sidx, didx, b0, b1, b2, b3, acc_sh,
                sg0, sg1, sg2, sg3, ss0, ss1, ss2, ss3):
        c = lax.axis_index("c")
        s = lax.axis_index("s")

        @pl.when(s < NS - 1)
        def _():
            pltpu.sync_copy(zrows.at[pl.ds(0, _RPT)],
                            acc_sh.at[pl.ds(s * _RPT, _RPT)])

        @pl.when(s == NS - 1)
        def _():
            pltpu.sync_copy(zrows,
                            acc_sh.at[pl.ds((NS - 1) * _RPT, _RPT_LAST)])

        plsc.subcore_barrier()

        bufs = (b0, b1, b2, b3)
        gsems = (sg0, sg1, sg2, sg3)
        ssems = (ss0, ss1, ss2, ss3)

        def g_start(j, b):
            pltpu.async_copy(h.at[sidx.at[pl.ds(j * _CH, _CH)]],
                             bufs[b], gsems[b])

        def g_wait(j, b):
            pltpu.make_async_copy(h.at[sidx.at[pl.ds(j * _CH, _CH)]],
                                  bufs[b], gsems[b]).wait()

        def s_start(j, b):
            pltpu.async_copy(bufs[b], acc_sh.at[didx.at[j]], ssems[b],
                             add=True)

        def s_wait(j, b):
            pltpu.make_async_copy(bufs[b], acc_sh.at[didx.at[j]],
                                  ssems[b]).wait()

        def pass_body(p, carry):
            pltpu.sync_copy(
                src.at[pl.ds(c * _EPC + s * _EPT + p * (_PP * _CH), _PP * _CH)],
                sidx)
            pltpu.sync_copy(dst5.at[c, s, p], didx)

            for b in range(4):
                g_start(b, b)

            def quad(t, carry2):
                j = 4 * t
                for b in range(4):
                    @pl.when(j + b < _PP)
                    def _(b=b):
                        g_wait(j + b, b)
                        s_start(j + b, b)
                for b in range(4):
                    @pl.when(j + 4 + b < _PP)
                    def _(b=b):
                        s_wait(j + b, b)
                        g_start(j + 4 + b, b)
                return carry2

            lax.fori_loop(0, (_PP + 3) // 4, quad, 0)

            for b in range(4):
                s_wait(0, b)
            return carry

        lax.fori_loop(0, _NPASS, pass_body, 0)

        plsc.subcore_barrier()

        @pl.when(s < NS - 1)
        def _():
            pltpu.sync_copy(acc_sh.at[pl.ds(s * _RPT, _RPT)],
                            out.at[c].at[pl.ds(s * _RPT, _RPT)])

        @pl.when(s == NS - 1)
        def _():
            pltpu.sync_copy(acc_sh.at[pl.ds((NS - 1) * _RPT, _RPT_LAST)],
                            out.at[c].at[pl.ds((NS - 1) * _RPT, _RPT_LAST)])

    return _sc_agg


_GCH = 32


@functools.lru_cache(maxsize=None)
def _get_sc_mask_gather():
    mesh = plsc.VectorSubcoreMesh(core_axis_name="c", subcore_axis_name="s")

    @functools.partial(
        pl.kernel,
        mesh=mesh,
        out_type=[jax.ShapeDtypeStruct((NMASK, D), _f32),
                  jax.ShapeDtypeStruct((NMASK, D), _f32)],
        scratch_types=[
            pltpu.VMEM((_GCH,), jnp.int32),
            pltpu.VMEM((_GCH, D), _f32),
            pltpu.VMEM((_GCH, D), _f32),
            pltpu.SemaphoreType.DMA,
        ],
    )
    def _sc_mask_gather(h_on, h_tg, mask, out_a, out_b, idx, ra, rb, sem):
        c = lax.axis_index("c")
        s = lax.axis_index("s")
        w = s * NC + c
        base = jnp.where(w * _GCH + _GCH <= NMASK, w * _GCH, NMASK - _GCH)
        pltpu.sync_copy(mask.at[pl.ds(base, _GCH)], idx)
        pltpu.async_copy(h_on.at[idx], ra, sem).wait()
        pltpu.async_copy(h_tg.at[idx], rb, sem).wait()
        pltpu.sync_copy(ra, out_a.at[pl.ds(base, _GCH)])
        pltpu.sync_copy(rb, out_b.at[pl.ds(base, _GCH)])

    return _sc_mask_gather


_RM = 1000


def _mask_body(x_ref, mask_ref, tok_ref, o_ref):
    i = pl.program_id(0)
    rows = lax.broadcasted_iota(jnp.int32, (_RM, 1), 0) + i * _RM
    hit = jnp.any(rows == mask_ref[...], axis=1, keepdims=True)
    o_ref[...] = jnp.where(hit, tok_ref[...], x_ref[...])


def _mask_call(x, mask_nodes, mask_token):
    return pl.pallas_call(
        _mask_body,
        grid=(N // _RM,),
        in_specs=[
            pl.BlockSpec((_RM, D), lambda i: (i, 0)),
            pl.BlockSpec((1, NMASK), lambda i: (0, 0)),
            pl.BlockSpec((1, D), lambda i: (0, 0)),
        ],
        out_specs=pl.BlockSpec((_RM, D), lambda i: (i, 0)),
        out_shape=jax.ShapeDtypeStruct((N, D), _f32),
    )(x, mask_nodes.reshape(1, NMASK), mask_token)


_R = 1000
_G = N // _R


def _stage_a_body(h, a0, a1, w, z, s1, s2):
    i = pl.program_id(0)

    @pl.when(i == 0)
    def _():
        s1[...] = jnp.zeros_like(s1)
        s2[...] = jnp.zeros_like(s2)

    m = h[...] + a0[0] + a1[0]
    zv = jnp.dot(m, w[...], preferred_element_type=_f32, precision=_HIGH)
    z[...] = zv
    s1[...] += jnp.sum(zv, axis=0, keepdims=True)
    s2[...] += jnp.sum(zv * zv, axis=0, keepdims=True)


def _stage_a(h, agg, w):
    outs = pl.pallas_call(
        _stage_a_body,
        grid=(_G,),
        in_specs=[
            pl.BlockSpec((_R, HID), lambda i: (i, 0)),
            pl.BlockSpec((1, _R, HID), lambda i: (0, i, 0)),
            pl.BlockSpec((1, _R, HID), lambda i: (1, i, 0)),
            pl.BlockSpec((HID, OUT), lambda i: (0, 0)),
        ],
        out_specs=[
            pl.BlockSpec((_R, OUT), lambda i: (i, 0)),
            pl.BlockSpec((1, OUT), lambda i: (0, 0)),
            pl.BlockSpec((1, OUT), lambda i: (0, 0)),
        ],
        out_shape=[
            jax.ShapeDtypeStruct((N, OUT), _f32),
            jax.ShapeDtypeStruct((1, OUT), _f32),
            jax.ShapeDtypeStruct((1, OUT), _f32),
        ],
        compiler_params=pltpu.CompilerParams(
            dimension_semantics=("arbitrary",)),
    )(h, agg, agg, w)
    return outs


def _bn_scale_shift(s1, s2, g, b, n):
    mu = s1 / n
    var = s2 / n - mu * mu
    scale = g / jnp.sqrt(var + 1e-5)
    shift = b - mu * scale
    return scale, shift


def _stage_b_body(z, s1, s2, g, b, w, y, t1, t2):
    i = pl.program_id(0)

    @pl.when(i == 0)
    def _():
        t1[...] = jnp.zeros_like(t1)
        t2[...] = jnp.zeros_like(t2)

    scale, shift = _bn_scale_shift(s1[...], s2[...], g[...], b[...], N)
    zn = jnp.maximum(z[...] * scale + shift, 0.0)
    yv = jnp.dot(zn, w[...], preferred_element_type=_f32, precision=_HIGH)
    y[...] = yv
    t1[...] += jnp.sum(yv, axis=0, keepdims=True)
    t2[...] += jnp.sum(yv * yv, axis=0, keepdims=True)


def _stage_b(z, s1, s2, g, b, w):
    vec = lambda: pl.BlockSpec((1, OUT), lambda i: (0, 0))
    outs = pl.pallas_call(
        _stage_b_body,
        grid=(_G,),
        in_specs=[
            pl.BlockSpec((_R, OUT), lambda i: (i, 0)),
            vec(), vec(), vec(), vec(),
            pl.BlockSpec((OUT, HID), lambda i: (0, 0)),
        ],
        out_specs=[
            pl.BlockSpec((_R, HID), lambda i: (i, 0)),
            pl.BlockSpec((1, HID), lambda i: (0, 0)),
            pl.BlockSpec((1, HID), lambda i: (0, 0)),
        ],
        out_shape=[
            jax.ShapeDtypeStruct((N, HID), _f32),
            jax.ShapeDtypeStruct((1, HID), _f32),
            jax.ShapeDtypeStruct((1, HID), _f32),
        ],
        compiler_params=pltpu.CompilerParams(
            dimension_semantics=("arbitrary",)),
    )(z, s1, s2, g, b, w)
    return outs


def _stage_c_body(y, t1, t2, g, b, h):
    scale, shift = _bn_scale_shift(t1[...], t2[...], g[...], b[...], N)
    h[...] = jnp.maximum(y[...] * scale + shift, 0.0)


def _stage_c(y, t1, t2, g, b):
    vec = lambda: pl.BlockSpec((1, HID), lambda i: (0, 0))
    return pl.pallas_call(
        _stage_c_body,
        grid=(_G,),
        in_specs=[
            pl.BlockSpec((_R, HID), lambda i: (i, 0)),
            vec(), vec(), vec(), vec(),
        ],
        out_specs=pl.BlockSpec((_R, HID), lambda i: (i, 0)),
        out_shape=jax.ShapeDtypeStruct((N, HID), _f32),
    )(y, t1, t2, g, b)


def _loss_body(a_ref, b_ref, o_ref):
    a = a_ref[...]
    b = b_ref[...]
    ab = jnp.sum(a * b, axis=1, keepdims=True)
    na = jnp.sqrt(jnp.sum(a * a, axis=1, keepdims=True)) + 1e-12
    nb = jnp.sqrt(jnp.sum(b * b, axis=1, keepdims=True)) + 1e-12
    t = 1.0 - ab / (na * nb)
    o_ref[0, 0] = jnp.sum(t) * (1.0 / NMASK)


def _loss_call(a, b):
    out = pl.pallas_call(
        _loss_body,
        in_specs=[
            pl.BlockSpec((NMASK, D), lambda: (0, 0)),
            pl.BlockSpec((NMASK, D), lambda: (0, 0)),
        ],
        out_specs=pl.BlockSpec(memory_space=pltpu.SMEM),
        out_shape=jax.ShapeDtypeStruct((1, 1), _f32),
    )(a, b)
    return out


def _dense(h, agg, W1, W2, G1, B1, Go, Bo):
    z, s1, s2 = _stage_a(h, agg, W1)
    y, t1, t2 = _stage_b(z, s1, s2,
                         G1.reshape(1, OUT), B1.reshape(1, OUT), W2)
    return _stage_c(y, t1, t2, Go.reshape(1, HID), Bo.reshape(1, HID))


def kernel(x, edge_index, mask_nodes, mask_token,
           onW1, onW2, onG1, onB1, onGo, onBo,
           tgW1, tgW2, tgG1, tgB1, tgGo, tgBo):
    src = edge_index[0]
    dst = edge_index[1]
    dst5 = dst.reshape(NC, NS, _NPASS, _PP, _CH)
    zrows = jnp.zeros((_RPT_LAST, D), _f32)

    sc_agg = _get_sc_agg()
    hA = _mask_call(x, mask_nodes, mask_token)
    hB = x

    for i in range(NLAYER):
        aggA = sc_agg(hA, src, dst5, zrows)
        aggB = sc_agg(hB, src, dst5, zrows)
        hA = _dense(hA, aggA, onW1[i], onW2[i], onG1[i], onB1[i],
                    onGo[i], onBo[i])
        hB = _dense(hB, aggB, tgW1[i], tgW2[i], tgG1[i], tgB1[i],
                    tgGo[i], tgBo[i])

    a, b = _get_sc_mask_gather()(hA, hB, mask_nodes)
    loss = _loss_call(a, b)
    return jnp.reshape(loss, ())
